# baseline (speedup 1.0000x reference)
import math

import jax
import jax.numpy as jnp
from jax.experimental import pallas as pl
from jax.experimental.pallas import tpu as pltpu

N_J = 100
N_M = 50
N_NODES = N_J * N_M
S = 65
INPUT_DIM = 2
SCEN_DIM = 64
HID = 64
NUM_INDS = 8
NUM_HEADS = 4
NUM_GNN_LAYERS = 3
BN_EPS = 1e-5


_ROW_BLK = 1000


def _adj_matmul_body(a_ref, h_ref, o_ref):
    o_ref[...] = jnp.dot(a_ref[...], h_ref[...],
                         preferred_element_type=jnp.float32)


def _adj_matmul(adj, h):
    n, f = h.shape
    grid = (N_NODES // _ROW_BLK,)
    return pl.pallas_call(
        _adj_matmul_body,
        grid=grid,
        in_specs=[
            pl.BlockSpec((_ROW_BLK, N_NODES), lambda i: (i, 0)),
            pl.BlockSpec((N_NODES, f), lambda i: (0, 0)),
        ],
        out_specs=pl.BlockSpec((_ROW_BLK, f), lambda i: (i, 0)),
        out_shape=jax.ShapeDtypeStruct((N_NODES, f), jnp.float32),
    )(adj, h)


def _lin(p, x):
    return x @ p['W'].T + p['b']


def _bn(p, x):
    mu = x.mean(axis=0)
    var = x.var(axis=0)
    return p['gamma'] * (x - mu) / jnp.sqrt(var + BN_EPS) + p['beta']


def _mab(p, Q, K):
    dv = p['fc_q']['W'].shape[0]
    Qp = _lin(p['fc_q'], Q)
    Kp = _lin(p['fc_k'], K)
    Vp = _lin(p['fc_v'], K)

    def split_heads(X):
        return jnp.concatenate(jnp.split(X, NUM_HEADS, axis=2), axis=0)

    Q_, K_, V_ = split_heads(Qp), split_heads(Kp), split_heads(Vp)
    A = jax.nn.softmax(jnp.einsum('bld,bmd->blm', Q_, K_) / math.sqrt(dv), axis=2)
    O = Q_ + jnp.einsum('blm,bmd->bld', A, V_)
    O = jnp.concatenate(jnp.split(O, NUM_HEADS, axis=0), axis=2)
    O = O + jax.nn.relu(_lin(p['fc_o'], O))
    return O


def _isab(p, X):
    B = X.shape[0]
    I = jnp.broadcast_to(p['I'], (B, p['I'].shape[1], p['I'].shape[2]))
    H = _mab(p['mab0'], I, X)
    return _mab(p['mab1'], X, H)


def _gin_mlp(p, x):
    h = x
    for i in range(len(p['linears']) - 1):
        h = jax.nn.relu(_bn(p['bns'][i], _lin(p['linears'][i], h)))
    return _lin(p['linears'][-1], h)


def _tanh_mlp(ps, x):
    h = x
    for i in range(len(ps) - 1):
        h = jnp.tanh(_lin(ps[i], h))
    return _lin(ps[-1], h)


def kernel(x, graph_pool, adj, params, padded_nei, candidate, mask):
    s = x.shape[0]
    xr = x.reshape(-1, s, INPUT_DIM)[:, 1:, :]
    scn = _lin(params['scenario_linear'], xr)
    scn = _isab(params['spm']['isab2'], _isab(params['spm']['isab1'], scn))
    scn_emb = scn.mean(axis=1)
    xc = jnp.concatenate([x[0], scn_emb], axis=-1)

    h = xc
    gnn = params['gnn']
    for layer in range(NUM_GNN_LAYERS - 1):
        pooled = _adj_matmul(adj, h)
        pooled_rep = _gin_mlp(gnn['mlps'][layer], pooled)
        h = jax.nn.relu(_bn(gnn['bns'][layer], pooled_rep))
    h_nodes = h
    h_pooled = graph_pool @ h

    b = candidate.shape[0]
    hdim = h_nodes.shape[-1]
    h_r = h_nodes.reshape(b, -1, hdim)
    dummy = jnp.broadcast_to(candidate[:, :, None], (b, candidate.shape[1], hdim))
    candidate_feature = jnp.take_along_axis(h_r, dummy, axis=1)
    h_pooled_rep = jnp.broadcast_to(h_pooled[:, None, :], candidate_feature.shape)
    concate_fea = jnp.concatenate([candidate_feature, h_pooled_rep], axis=-1)
    scores = _tanh_mlp(params['actor'], concate_fea)
    mask_r = mask.reshape(scores.shape)
    scores = jnp.where(mask_r, -jnp.inf, scores)
    pi = jax.nn.softmax(scores, axis=1)
    v = _tanh_mlp(params['critic'], h_pooled)
    return pi, v
